# baseline (speedup 1.0000x reference)
import jax
import jax.numpy as jnp
from jax import lax
from jax.experimental import pallas as pl
from jax.experimental.pallas import tpu as pltpu
from jax.experimental.pallas import tpu_sc as plsc

NC = 2
NS = 16
L = 16
NW = NC * NS
B = 16384
D = 32
BPW = B // NW
NBLK = 7813
WBLK = 2
NWAVE = (NBLK - 1) // WBLK
WPT = (NWAVE + NW - 1) // NW
TAILTILE = NWAVE % NW
CAP = B + L
NCHUNK = 8
CHUNK = B // NCHUNK
STG = B + 8
SROW = 128


def _sweep_body(users_hbm, items_hbm, utT_hbm, itT_hbm, params_hbm,
                stagu_hbm, stagi_hbm,
                chunk_v, selid_v, selpos_v, scal_ref,
                ublk_v, iblk_v, pack_v, pos_v, params_v,
                sem_c, sem_w0, sem_w1, sem_s00, sem_s01, sem_s10, sem_s11):
    wid = lax.axis_index("s") * NC + lax.axis_index("c")

    pltpu.sync_copy(params_hbm, params_v)
    w_lo = params_v[pl.ds(0, L)]
    w_hi = params_v[pl.ds(L, L)]
    ones = jnp.ones((L,), jnp.float32)
    dlane = lax.iota(jnp.int32, L)

    tbls = (utT_hbm, itT_hbm)
    bufs = (ublk_v, iblk_v)
    stags = (stagu_hbm, stagi_hbm)
    wvecs = ((w_lo, w_hi), (ones, ones))
    ssems = ((sem_s00, sem_s01), (sem_s10, sem_s11))

    for t, src in ((0, users_hbm), (1, items_hbm)):

        def chunk_step(c, cnt, t=t, src=src):
            pltpu.async_copy(src.at[pl.ds(c * CHUNK, CHUNK)], chunk_v,
                             sem_c).wait()

            def vreg_step(q, cnt2, t=t, c=c):
                ids = chunk_v[pl.ds(q * L, L)]
                mask = lax.bitwise_and(lax.shift_right_logical(ids, 8),
                                       NW - 1) == wid
                npop = plsc.all_reduce_population_count(mask)[0]

                @pl.when(npop > 0)
                def _():
                    pos = c * CHUNK + q * L + dlane
                    mi = mask.astype(jnp.int32)
                    pref = plsc.cumsum(mi) - mi
                    slots = cnt2 + pref
                    tv = jnp.full((L,), t, jnp.int32)
                    plsc.store_scatter(selid_v, [tv, slots], ids,
                                       mask=mask)
                    plsc.store_scatter(selpos_v, [tv, slots], pos,
                                       mask=mask)

                return cnt2 + npop

            return lax.fori_loop(0, CHUNK // L, vreg_step, cnt)

        scal_ref[0, t] = lax.fori_loop(0, NCHUNK, chunk_step, jnp.int32(0))
        scal_ref[1, t] = jnp.int32(0)

    cnts = (scal_ref[0, 0], scal_ref[0, 1])

    wsems = (sem_w0, sem_w1)

    def issue_wave(widx, dbuf, pw):
        for t in range(2):
            for j in range(WBLK):
                off = pl.multiple_of((widx * WBLK + j) * 128, 128)
                pltpu.async_copy(tbls[t].at[:, pl.ds(off, 128)],
                                 bufs[t].at[dbuf, j], wsems[pw])

    def drain_wave(dbuf, pw):
        for t in range(2):
            for j in range(WBLK):
                pltpu.make_async_copy(utT_hbm.at[:, pl.ds(0, 128)],
                                      bufs[t].at[dbuf, j], wsems[pw]).wait()

    def extract_wave(widx, dbuf):
        for t in range(2):
            wlo, whi = wvecs[t]

            def q_step(q, carry, t=t):
                sids = selid_v[t, pl.ds(q * L, L)]
                sposv = selpos_v[t, pl.ds(q * L, L)]
                valid = (q * L + dlane) < cnts[t]
                mask = jnp.logical_and(
                    lax.shift_right_logical(sids, 8) == widx, valid)
                npop = plsc.all_reduce_population_count(mask)[0]

                @pl.when(npop > 0)
                def _(t=t):
                    nsc = scal_ref[1, t]
                    pb = lax.rem(nsc, jnp.int32(2))

                    @pl.when(nsc >= 2)
                    def _(t=t):
                        @pl.when(pb == 0)
                        def _():
                            pltpu.make_async_copy(
                                pack_v.at[t, 0], stags[t].at[pos_v.at[t, 0]],
                                ssems[t][0]).wait()

                        @pl.when(pb == 1)
                        def _():
                            pltpu.make_async_copy(
                                pack_v.at[t, 1], stags[t].at[pos_v.at[t, 1]],
                                ssems[t][1]).wait()

                    for pbv in range(2):
                        @pl.when(pb == pbv)
                        def _(pbv=pbv, t=t):
                            pos_v[t, pbv] = jnp.full((L,), B, jnp.int32)
                            mi = mask.astype(jnp.int32)
                            pref = plsc.cumsum(mi) - mi
                            tv = jnp.full((L,), t, jnp.int32)
                            pv = jnp.full((L,), pbv, jnp.int32)
                            plsc.store_scatter(pos_v, [tv, pv, pref],
                                               sposv, mask=mask)
                            for j in range(L):
                                @pl.when(mi[j] == 1)
                                def _(j=j, t=t, pbv=pbv):
                                    idj = sids[j]
                                    col = jnp.full(
                                        (L,), lax.bitwise_and(idj, 127),
                                        jnp.int32)
                                    blkv = jnp.full(
                                        (L,),
                                        lax.bitwise_and(
                                            lax.shift_right_logical(idj, 7),
                                            WBLK - 1), jnp.int32)
                                    bufv = jnp.full((L,), dbuf, jnp.int32)
                                    lo = plsc.load_gather(
                                        bufs[t], [bufv, blkv, dlane, col])
                                    hi = plsc.load_gather(
                                        bufs[t], [bufv, blkv, dlane + L, col])
                                    prev = pref[j]
                                    pack_v[t, pbv, prev, pl.ds(0, L)] = (
                                        lo * wlo)
                                    pack_v[t, pbv, prev, pl.ds(L, L)] = (
                                        hi * whi)

                            pltpu.async_copy(pack_v.at[t, pbv],
                                             stags[t].at[pos_v.at[t, pbv]],
                                             ssems[t][pbv])

                    scal_ref[1, t] = nsc + 1

                return carry

            nq = lax.div(cnts[t] + (L - 1), jnp.int32(L))
            lax.fori_loop(0, nq, q_step, 0)

    issue_wave(wid, 0, 0)

    def guarded_wave(it, carry):
        @pl.when(it * NW + wid < NWAVE)
        def _():
            widx = it * NW + wid
            nxt = widx + NW

            @pl.when(nxt < NWAVE)
            def _():
                for pwv in range(2):
                    @pl.when(lax.rem(it + 1, 2) == pwv)
                    def _(pwv=pwv):
                        issue_wave(nxt, lax.rem(it + 1, 2), pwv)

            for pwv in range(2):
                @pl.when(lax.rem(it, 2) == pwv)
                def _(pwv=pwv):
                    drain_wave(lax.rem(it, 2), pwv)
            extract_wave(widx, lax.rem(it, 2))

        return carry

    lax.fori_loop(0, WPT, guarded_wave, 0)

    @pl.when(wid == TAILTILE)
    def _():
        for t in range(2):
            off = pl.multiple_of(NWAVE * WBLK * 128, 128)
            pltpu.async_copy(tbls[t].at[:, pl.ds(off, 128)],
                             bufs[t].at[0, 0], sem_w0).wait()
        extract_wave(jnp.int32(NWAVE), 0)

    for t in range(2):
        for pbv in range(2):
            nsc = scal_ref[1, t]
            pend = jnp.logical_or(
                jnp.logical_and(nsc >= 1,
                                lax.rem(nsc - 1, jnp.int32(2)) == pbv),
                nsc >= 2)

            @pl.when(pend)
            def _(t=t, pbv=pbv):
                pltpu.make_async_copy(pack_v.at[t, pbv],
                                      stags[t].at[pos_v.at[t, pbv]],
                                      ssems[t][pbv]).wait()


def _join_body(stagu_hbm, stagi_hbm, params_hbm, out_hbm,
               u_v, i_v, out_v, params_v, sem_u, sem_i):
    wid = lax.axis_index("s") * NC + lax.axis_index("c")
    base = wid * BPW

    pltpu.sync_copy(params_hbm, params_v)
    bias = params_v[pl.ds(2 * L, L)][0]
    dlane = lax.iota(jnp.int32, L)

    CH = 128

    def chunk_step(c, carry):
        r0 = pl.multiple_of(base + c * CH, CH)
        cu = pltpu.async_copy(stagu_hbm.at[pl.ds(r0, CH)], u_v, sem_u)
        ci = pltpu.async_copy(stagi_hbm.at[pl.ds(r0, CH)], i_v, sem_i)
        cu.wait()
        ci.wait()

        def group(g, carry2):
            rowv = g * L + dlane
            acc = jnp.zeros((L,), jnp.float32)
            for col in range(D):
                cv = jnp.full((L,), col, jnp.int32)
                uu = plsc.load_gather(u_v, [rowv, cv])
                vv = plsc.load_gather(i_v, [rowv, cv])
                acc = acc + uu * vv
            z = acc + bias
            out_v[pl.ds((c * CH + g * L), L)] = 1.0 / (1.0 + jnp.exp(-z))
            return carry2

        lax.fori_loop(0, CH // L, group, 0)
        return carry

    lax.fori_loop(0, BPW // CH, chunk_step, 0)
    pltpu.sync_copy(out_v, out_hbm.at[pl.ds(base, BPW)])


@jax.jit
def _gmf(users, items, user_table_t, item_table_t, params):
    mesh = plsc.VectorSubcoreMesh(core_axis_name="c", subcore_axis_name="s",
                                  num_cores=NC, num_subcores=NS)
    stagu, stagi = pl.kernel(
        _sweep_body,
        out_type=(jax.ShapeDtypeStruct((STG, SROW), jnp.float32),
                  jax.ShapeDtypeStruct((STG, SROW), jnp.float32)),
        mesh=mesh,
        compiler_params=pltpu.CompilerParams(needs_layout_passes=False),
        scratch_types=[
            pltpu.VMEM((CHUNK,), jnp.int32),
            pltpu.VMEM((2, CAP), jnp.int32),
            pltpu.VMEM((2, CAP), jnp.int32),
            pltpu.SMEM((2, 2), jnp.int32),
            pltpu.VMEM((2, WBLK, D, 128), jnp.float32),
            pltpu.VMEM((2, WBLK, D, 128), jnp.float32),
            pltpu.VMEM((2, 2, L, SROW), jnp.float32),
            pltpu.VMEM((2, 2, L), jnp.int32),
            pltpu.VMEM((D + L,), jnp.float32),
            pltpu.SemaphoreType.DMA,
            pltpu.SemaphoreType.DMA,
            pltpu.SemaphoreType.DMA,
            pltpu.SemaphoreType.DMA,
            pltpu.SemaphoreType.DMA,
            pltpu.SemaphoreType.DMA,
            pltpu.SemaphoreType.DMA,
        ],
    )(users, items, user_table_t, item_table_t, params)

    return pl.kernel(
        _join_body,
        out_type=jax.ShapeDtypeStruct((B,), jnp.float32),
        mesh=mesh,
        compiler_params=pltpu.CompilerParams(needs_layout_passes=False),
        scratch_types=[
            pltpu.VMEM((128, SROW), jnp.float32),
            pltpu.VMEM((128, SROW), jnp.float32),
            pltpu.VMEM((BPW,), jnp.float32),
            pltpu.VMEM((D + L,), jnp.float32),
            pltpu.SemaphoreType.DMA,
            pltpu.SemaphoreType.DMA,
        ],
    )(stagu, stagi, params)


def kernel(users, items, user_table, item_table, W, b):
    params = jnp.concatenate(
        [W.reshape(-1), b.reshape(-1),
         jnp.zeros((L - 1,), jnp.float32)]).astype(jnp.float32)
    return _gmf(users.astype(jnp.int32), items.astype(jnp.int32),
                user_table.T, item_table.T, params)


# speedup vs baseline: 36.1496x; 36.1496x over previous
import jax
import jax.numpy as jnp
from jax import lax
from jax.experimental import pallas as pl
from jax.experimental.pallas import tpu as pltpu
from jax.experimental.pallas import tpu_sc as plsc

NC = 2
NS = 16
L = 16
NW = NC * NS
B = 16384
D = 32
BPW = B // NW
SW = 16
NSW = BPW // SW
WIDS = 4
NSUB = SW // WIDS
BLK = 128


def _gmf_body(users_hbm, items_hbm, utT_hbm, itT_hbm, params_hbm, out_hbm,
              uidx_v, iidx_v, ublk_v, iblk_v, psums_v, out_v,
              params_v, sem_p, sem_a, sem_b):
    wid = lax.axis_index("s") * NC + lax.axis_index("c")
    base = wid * BPW

    pltpu.sync_copy(users_hbm.at[pl.ds(base, BPW)], uidx_v)
    pltpu.sync_copy(items_hbm.at[pl.ds(base, BPW)], iidx_v)
    cp = pltpu.async_copy(params_hbm, params_v, sem_p)
    cp.wait()

    w_lo = params_v[pl.ds(0, L)]
    w_hi = params_v[pl.ds(L, L)]
    bias = params_v[pl.ds(2 * L, L)][0]
    dlane = lax.iota(jnp.int32, L)

    sems = (sem_a, sem_b)

    def issue_sub(us, is_, sub):
        buf, sem = sub % 2, sems[sub % 2]
        for j in range(WIDS):
            ub = lax.shift_right_logical(us[sub * WIDS + j], 7)
            ib = lax.shift_right_logical(is_[sub * WIDS + j], 7)
            pltpu.async_copy(
                utT_hbm.at[:, pl.ds(pl.multiple_of(ub * BLK, BLK), BLK)],
                ublk_v.at[buf, j], sem)
            pltpu.async_copy(
                itT_hbm.at[:, pl.ds(pl.multiple_of(ib * BLK, BLK), BLK)],
                iblk_v.at[buf, j], sem)

    def drain_sub(sub):
        buf, sem = sub % 2, sems[sub % 2]
        for j in range(WIDS):
            pltpu.make_async_copy(utT_hbm.at[:, pl.ds(0, BLK)],
                                  ublk_v.at[buf, j], sem).wait()
            pltpu.make_async_copy(itT_hbm.at[:, pl.ds(0, BLK)],
                                  iblk_v.at[buf, j], sem).wait()

    def extract_sub(s, us, is_, sub):
        buf = sub % 2
        bufv = jnp.full((L,), buf, jnp.int32)
        for j in range(WIDS):
            k = s * SW + sub * WIDS + j
            jv = jnp.full((L,), j, jnp.int32)
            ucv = jnp.full((L,), lax.bitwise_and(us[sub * WIDS + j], 127), jnp.int32)
            icv = jnp.full((L,), lax.bitwise_and(is_[sub * WIDS + j], 127), jnp.int32)
            u_lo = plsc.load_gather(ublk_v, [bufv, jv, dlane, ucv])
            u_hi = plsc.load_gather(ublk_v, [bufv, jv, dlane + L, ucv])
            v_lo = plsc.load_gather(iblk_v, [bufv, jv, dlane, icv])
            v_hi = plsc.load_gather(iblk_v, [bufv, jv, dlane + L, icv])
            psum = (u_lo * v_lo) * w_lo + (u_hi * v_hi) * w_hi
            psums_v[pl.ds(k * L, L)] = psum

    def superwave(s, carry):
        uv = uidx_v[pl.ds(s * SW, SW)]
        iv = iidx_v[pl.ds(s * SW, SW)]
        us = [uv[j] for j in range(SW)]
        is_ = [iv[j] for j in range(SW)]
        issue_sub(us, is_, 0)
        issue_sub(us, is_, 1)
        for sub in range(NSUB):
            drain_sub(sub)
            extract_sub(s, us, is_, sub)
            if sub + 2 < NSUB:
                issue_sub(us, is_, sub + 2)
        return carry

    lax.fori_loop(0, NSW, superwave, 0)

    lane16 = dlane * L

    def reduce_group(g, carry):
        acc = jnp.zeros((L,), jnp.float32)
        for c in range(L):
            acc = acc + plsc.load_gather(psums_v, [g * (L * L) + lane16 + c])
        z = acc + bias
        out_v[pl.ds(g * L, L)] = 1.0 / (1.0 + jnp.exp(-z))
        return carry

    lax.fori_loop(0, BPW // L, reduce_group, 0)
    pltpu.sync_copy(out_v, out_hbm.at[pl.ds(base, BPW)])


@jax.jit
def _gmf(users, items, user_table_t, item_table_t, params):
    mesh = plsc.VectorSubcoreMesh(core_axis_name="c", subcore_axis_name="s",
                                  num_cores=NC, num_subcores=NS)
    return pl.kernel(
        _gmf_body,
        out_type=jax.ShapeDtypeStruct((B,), jnp.float32),
        mesh=mesh,
        compiler_params=pltpu.CompilerParams(needs_layout_passes=False),
        scratch_types=[
            pltpu.VMEM((BPW,), jnp.int32),
            pltpu.VMEM((BPW,), jnp.int32),
            pltpu.VMEM((2, WIDS, D, BLK), jnp.float32),
            pltpu.VMEM((2, WIDS, D, BLK), jnp.float32),
            pltpu.VMEM((BPW * L,), jnp.float32),
            pltpu.VMEM((BPW,), jnp.float32),
            pltpu.VMEM((D + L,), jnp.float32),
            pltpu.SemaphoreType.DMA,
            pltpu.SemaphoreType.DMA,
            pltpu.SemaphoreType.DMA,
        ],
    )(users, items, user_table_t, item_table_t, params)


def kernel(users, items, user_table, item_table, W, b):
    params = jnp.concatenate(
        [W.reshape(-1), b.reshape(-1),
         jnp.zeros((L - 1,), jnp.float32)]).astype(jnp.float32)
    return _gmf(users.astype(jnp.int32), items.astype(jnp.int32),
                user_table.T, item_table.T, params)
